# baseline (speedup 1.0000x reference)
import functools
import jax
import jax.numpy as jnp
from jax import lax
from jax.experimental import pallas as pl
from jax.experimental.pallas import tpu as pltpu
from jax.experimental.pallas import tpu_sc as plsc

NC = 2
NS = 16
L = 16
NW = NC * NS
B = 16384
D = 64
V = 1000000
BPW = B // NW
CHUNK = 128
HALFW = BPW // 2
NPASS = 2
RC = 4096
TGRID = (V + RC - 1) // RC
VPAD = TGRID * RC

_mesh = plsc.VectorSubcoreMesh(core_axis_name="c", subcore_axis_name="s")


def _tr_body(u_ref, i_ref, out_ref):
    tu = jnp.swapaxes(u_ref[...], 0, 1)
    ti = jnp.swapaxes(i_ref[...], 0, 1)
    out_ref[...] = jnp.concatenate([tu, ti], axis=1)


_tc_combine = pl.pallas_call(
    _tr_body,
    grid=(TGRID,),
    in_specs=[
        pl.BlockSpec((D, RC), lambda i: (0, i)),
        pl.BlockSpec((D, RC), lambda i: (0, i)),
    ],
    out_specs=pl.BlockSpec((RC, 2 * D), lambda i: (i, 0)),
    out_shape=jax.ShapeDtypeStruct((VPAD, 2 * D), jnp.float32),
)


@functools.partial(
    pl.kernel,
    out_type=jax.ShapeDtypeStruct((B,), jnp.float32),
    mesh=_mesh,
    compiler_params=pltpu.CompilerParams(needs_layout_passes=False),
    scratch_types=[
        pltpu.VMEM((BPW // CHUNK, CHUNK), jnp.int32),
        pltpu.VMEM((BPW // CHUNK, CHUNK), jnp.int32),
        pltpu.VMEM((HALFW, 2 * D), jnp.float32),
        pltpu.VMEM((HALFW, 2 * D), jnp.float32),
        pltpu.VMEM((BPW,), jnp.float32),
        pltpu.SemaphoreType.DMA,
    ],
)
def _mf_rating(user_hbm, item_hbm, comb_hbm, out_hbm,
               uidx, iidx, urows, irows, out_v, gsem):
    wid = lax.axis_index("s") * NC + lax.axis_index("c")
    base = wid * BPW
    nchunk = BPW // CHUNK

    for c in range(nchunk):
        pltpu.sync_copy(user_hbm.at[pl.ds(base + c * CHUNK, CHUNK)],
                        uidx.at[c])
        pltpu.sync_copy(item_hbm.at[pl.ds(base + c * CHUNK, CHUNK)],
                        iidx.at[c])

    row_iota = lax.iota(jnp.int32, L)

    def do_pass(p):
        copies = []
        for cc in range(HALFW // CHUNK):
            c = p * (HALFW // CHUNK) + cc
            copies.append(pltpu.async_copy(
                comb_hbm.at[uidx.at[c]],
                urows.at[pl.ds(cc * CHUNK, CHUNK)], gsem))
            copies.append(pltpu.async_copy(
                comb_hbm.at[iidx.at[c]],
                irows.at[pl.ds(cc * CHUNK, CHUNK)], gsem))
        for cp in copies:
            cp.wait()

        def group(g, carry):
            idx_row = g * L + row_iota
            acc = jnp.zeros((L,), jnp.float32)
            for d in range(D):
                dvec = jnp.full((L,), d, jnp.int32)
                u = plsc.load_gather(urows, [idx_row, dvec])
                i = plsc.load_gather(irows, [idx_row, dvec + D])
                acc = acc + u * i
            out_v[pl.ds(p * HALFW + g * L, L)] = acc
            return carry

        lax.fori_loop(0, HALFW // L, group, 0)

    for p in range(NPASS):
        do_pass(p)

    pltpu.sync_copy(out_v, out_hbm.at[pl.ds(base, BPW)])


def kernel(user, item, user_emb, item_emb):
    comb = jnp.concatenate([user_emb, item_emb], axis=1)
    return _mf_rating(user, item, comb)


# speedup vs baseline: 1.9013x; 1.9013x over previous
import functools
import jax
import jax.numpy as jnp
from jax import lax
from jax.experimental import pallas as pl
from jax.experimental.pallas import tpu as pltpu
from jax.experimental.pallas import tpu_sc as plsc

NC = 2
NS = 16
L = 16
NW = NC * NS
B = 16384
D = 64
V = 1000000
BPW = B // NW
CHUNK = 128
HALFW = BPW // 2
NPASS = 2
RC = 8192
TGRID = (V + RC - 1) // RC
VPAD = TGRID * RC

_mesh = plsc.VectorSubcoreMesh(core_axis_name="c", subcore_axis_name="s")


def _tr_body(u_ref, i_ref, out_ref):
    tu = jnp.swapaxes(u_ref[...], 0, 1)
    ti = jnp.swapaxes(i_ref[...], 0, 1)
    out_ref[...] = jnp.concatenate([tu, ti], axis=1)


_tc_combine = pl.pallas_call(
    _tr_body,
    grid=(TGRID,),
    in_specs=[
        pl.BlockSpec((D, RC), lambda i: (0, i)),
        pl.BlockSpec((D, RC), lambda i: (0, i)),
    ],
    out_specs=pl.BlockSpec((RC, 2 * D), lambda i: (i, 0)),
    out_shape=jax.ShapeDtypeStruct((VPAD, 2 * D), jnp.float32),
    compiler_params=pltpu.CompilerParams(
        dimension_semantics=("parallel",)),
)


@functools.partial(
    pl.kernel,
    out_type=jax.ShapeDtypeStruct((B,), jnp.float32),
    mesh=_mesh,
    compiler_params=pltpu.CompilerParams(needs_layout_passes=False),
    scratch_types=[
        pltpu.VMEM((BPW // CHUNK, CHUNK), jnp.int32),
        pltpu.VMEM((BPW // CHUNK, CHUNK), jnp.int32),
        pltpu.VMEM((HALFW, 2 * D), jnp.float32),
        pltpu.VMEM((HALFW, 2 * D), jnp.float32),
        pltpu.VMEM((BPW,), jnp.float32),
        pltpu.SemaphoreType.DMA,
    ],
)
def _mf_rating(user_hbm, item_hbm, comb_hbm, out_hbm,
               uidx, iidx, urows, irows, out_v, gsem):
    wid = lax.axis_index("s") * NC + lax.axis_index("c")
    base = wid * BPW
    nchunk = BPW // CHUNK

    for c in range(nchunk):
        pltpu.sync_copy(user_hbm.at[pl.ds(base + c * CHUNK, CHUNK)],
                        uidx.at[c])
        pltpu.sync_copy(item_hbm.at[pl.ds(base + c * CHUNK, CHUNK)],
                        iidx.at[c])

    row_iota = lax.iota(jnp.int32, L)

    def do_pass(p):
        copies = []
        for cc in range(HALFW // CHUNK):
            c = p * (HALFW // CHUNK) + cc
            copies.append(pltpu.async_copy(
                comb_hbm.at[uidx.at[c]],
                urows.at[pl.ds(cc * CHUNK, CHUNK)], gsem))
            copies.append(pltpu.async_copy(
                comb_hbm.at[iidx.at[c]],
                irows.at[pl.ds(cc * CHUNK, CHUNK)], gsem))
        for cp in copies:
            cp.wait()

        def group(g, carry):
            idx_row = g * L + row_iota
            acc = jnp.zeros((L,), jnp.float32)
            for d in range(D):
                dvec = jnp.full((L,), d, jnp.int32)
                u = plsc.load_gather(urows, [idx_row, dvec])
                i = plsc.load_gather(irows, [idx_row, dvec + D])
                acc = acc + u * i
            out_v[pl.ds(p * HALFW + g * L, L)] = acc
            return carry

        lax.fori_loop(0, HALFW // L, group, 0)

    for p in range(NPASS):
        do_pass(p)

    pltpu.sync_copy(out_v, out_hbm.at[pl.ds(base, BPW)])


def kernel(user, item, user_emb, item_emb):
    comb = _tc_combine(user_emb.T, item_emb.T)
    return _mf_rating(user, item, comb)
